# baseline (speedup 1.0000x reference)
import jax
import jax.numpy as jnp
from jax import lax
from jax.experimental import pallas as pl
from jax.experimental.pallas import tpu as pltpu
from jax.experimental.pallas import tpu_sc as plsc

D = 128
K = 64
N = 8192
NC = 2
NS = 16
NW = NC * NS
CHUNK = N // NW
HALF = N // NC
BLK = 128
NBLK = N // BLK
BPC = NBLK // NC
SEGW = 80
PPC = 96
NSTEP = NC * PPC


def _bsearch_count_le(table_ref, q, zeros16, n):
    lo = zeros16
    step = n
    while step >= 1:
        cand = jnp.minimum(lo + step, n)
        val = plsc.load_gather(table_ref, [cand - 1])
        lo = jnp.where(val <= q, cand, lo)
        step //= 2
    return lo


def _route_body(u_hbm, x_hbm, part_hbm,
                xs_hbm, pos_hbm, pb_hbm, pe_hbm, plo_hbm, phi_hbm, hx_hbm,
                part_v, u_v, idx_v, base_v, hist_v, hall_v, pos2_v,
                tmpa_v, tmpb_v, segrow_v, blo_v, cumi_v, cumx_v, pair_v,
                xrows_v, sem, semx):
    c = lax.axis_index("c")
    s = lax.axis_index("s")
    wid = c * NS + s
    tok0 = wid * CHUNK

    xcopy = pltpu.async_copy(x_hbm.at[pl.ds(tok0, CHUNK)], xrows_v, semx)
    pltpu.sync_copy(part_hbm, part_v)
    pltpu.sync_copy(u_hbm.at[pl.ds(tok0, CHUNK)], u_v)

    iota = lax.iota(jnp.int32, 16)
    zeros16 = jnp.zeros((16,), jnp.int32)
    ones16 = jnp.ones((16,), jnp.int32)

    for kv in range(4):
        hist_v[pl.ds(kv * 16, 16)] = zeros16

    for i in range(CHUNK // 16):
        uq = u_v[pl.ds(i * 16, 16)]
        cnt = _bsearch_count_le(part_v, uq, zeros16, K)
        idx = jnp.minimum(cnt, K - 1)
        idx_v[pl.ds(i * 16, 16)] = idx
        plsc.addupdate_scatter(hist_v, [idx], ones16)

    pltpu.sync_copy(hist_v, hx_hbm.at[c, s])
    plsc.subcore_barrier()
    pltpu.sync_copy(hx_hbm.at[c], hall_v)

    svec = zeros16 + s
    tots = []
    mybs = []
    for kv in range(4):
        tot = zeros16
        myb = zeros16
        for t in range(NS):
            row = hall_v[t, pl.ds(kv * 16, 16)]
            tot = tot + row
            myb = myb + jnp.where((zeros16 + t) < svec, row, zeros16)
        tots.append(tot)
        mybs.append(myb)

    carry = zeros16
    core_off = (zeros16 + c) * HALF
    for kv in range(4):
        inc = plsc.cumsum(tots[kv])
        start = inc - tots[kv] + carry + core_off
        segrow_v[pl.ds(kv * 16, 16)] = start
        base_v[pl.ds(kv * 16, 16)] = start + mybs[kv]
        tmpa_v[...] = inc
        last = plsc.load_gather(tmpa_v, [zeros16 + 15])
        carry = carry + last
    endvec = jnp.where(iota == 0, (zeros16 + c + 1) * HALF, zeros16)
    segrow_v[pl.ds(64, 16)] = endvec

    @pl.when(s == 0)
    def _():
        row0_off = (zeros16 + c) * HALF
        nbs = []
        for bv in range(BPC // 16):
            bid = iota + bv * 16
            start_b = bid * BLK + row0_off
            cl = _bsearch_count_le(segrow_v, start_b, zeros16, K)
            cu = _bsearch_count_le(segrow_v, start_b + (BLK - 1), zeros16, K)
            cl = jnp.clip(cl, 1, K)
            cu = jnp.clip(cu, 1, K)
            blo_v[pl.ds(bv * 16, 16)] = cl - 1
            nbs.append(cu - cl + 1)
        inc0 = plsc.cumsum(nbs[0])
        tmpa_v[...] = inc0
        carry0 = plsc.load_gather(tmpa_v, [zeros16 + 15])
        inc1 = plsc.cumsum(nbs[1]) + carry0
        cumi_v[pl.ds(0, 16)] = inc0
        cumi_v[pl.ds(16, 16)] = inc1
        cumx_v[pl.ds(0, 16)] = inc0 - nbs[0]
        cumx_v[pl.ds(16, 16)] = inc1 - nbs[1]
        tmpa_v[...] = inc1
        total = plsc.load_gather(tmpa_v, [zeros16 + 15])
        for sv in range(PPC // 16):
            g = iota + sv * 16
            bb = _bsearch_count_le(cumi_v, g, zeros16, BPC)
            b_loc = jnp.minimum(bb, BPC - 1)
            pad = g >= total
            eoff = g - plsc.load_gather(cumx_v, [b_loc])
            e = jnp.clip(plsc.load_gather(blo_v, [b_loc]) + eoff, 0, K - 1)
            b_glob = b_loc + (zeros16 + c) * BPC
            row0 = b_glob * BLK
            lo = jnp.maximum(plsc.load_gather(segrow_v, [e]), row0)
            hi = jnp.minimum(plsc.load_gather(segrow_v, [e + 1]), row0 + BLK)
            lo = jnp.where(pad, zeros16, lo)
            hi = jnp.where(pad, zeros16, hi)
            pair_v[0, pl.ds(sv * 16, 16)] = b_glob
            pair_v[1, pl.ds(sv * 16, 16)] = e
            pair_v[2, pl.ds(sv * 16, 16)] = lo
            pair_v[3, pl.ds(sv * 16, 16)] = hi
        pltpu.sync_copy(pair_v.at[0], pb_hbm.at[pl.ds(c * PPC, PPC)])
        pltpu.sync_copy(pair_v.at[1], pe_hbm.at[pl.ds(c * PPC, PPC)])
        pltpu.sync_copy(pair_v.at[2], plo_hbm.at[pl.ds(c * PPC, PPC)])
        pltpu.sync_copy(pair_v.at[3], phi_hbm.at[pl.ds(c * PPC, PPC)])

    for i in range(CHUNK // 16):
        idx = idx_v[pl.ds(i * 16, 16)]
        sk, sv = plsc.sort_key_val(idx, iota)
        tmpa_v[...] = sk
        prev = plsc.load_gather(tmpa_v, [jnp.maximum(iota - 1, 0)])
        newf = jnp.logical_or(iota == 0, sk != prev)
        runstart = plsc.cummax(jnp.where(newf, iota, zeros16))
        rank_sorted = iota - runstart
        plsc.store_scatter(tmpb_v, [sv], rank_sorted)
        rank = tmpb_v[...]
        pos_vec = plsc.load_gather(base_v, [idx]) + rank
        plsc.addupdate_scatter(base_v, [idx], ones16)
        pos_vec = jnp.clip(pos_vec, 0, N - 1)
        pos2_v[i // 8, pl.ds((i % 8) * 16, 16)] = pos_vec

    pltpu.sync_copy(pos2_v, pos_hbm.at[pl.ds(2 * wid, 2)])
    xcopy.wait()
    d0 = pltpu.async_copy(xrows_v.at[pl.ds(0, 128)], xs_hbm.at[pos2_v.at[0]],
                          sem)
    d1 = pltpu.async_copy(xrows_v.at[pl.ds(128, 128)], xs_hbm.at[pos2_v.at[1]],
                          sem)
    d0.wait()
    d1.wait()


def _gemm_body(pb_ref, pe_ref, plo_ref, phi_ref, xs_ref, devs_ref, means_ref,
               out_ref):
    g = pl.program_id(0)
    bcur = pb_ref[g]
    prev = pb_ref[jnp.maximum(g - 1, 0)]
    first = jnp.logical_or(g == 0, bcur != prev)
    e = pe_ref[g]
    lo = plo_ref[g]
    hi = phi_ref[g]

    xb = xs_ref[...].astype(jnp.bfloat16)
    dk = devs_ref[e]
    prod = lax.dot_general(
        xb, dk,
        dimension_numbers=(((1,), (1,)), ((), ())),
        preferred_element_type=jnp.float32,
    )
    rows = bcur * BLK + lax.broadcasted_iota(jnp.int32, (BLK, 1), 0)
    m = jnp.logical_and(rows >= lo, rows < hi).astype(jnp.float32)
    contrib = m * (prod + means_ref[e].reshape(1, D))

    @pl.when(first)
    def _():
        out_ref[...] = jnp.zeros((BLK, D), jnp.float32)

    out_ref[...] = out_ref[...] + contrib


def _unsort_body(ys_hbm, pos_hbm, y_hbm, pos2_v, rows_v, sem):
    c = lax.axis_index("c")
    s = lax.axis_index("s")
    wid = c * NS + s
    pltpu.sync_copy(pos_hbm.at[pl.ds(2 * wid, 2)], pos2_v)
    d0 = pltpu.async_copy(ys_hbm.at[pos2_v.at[0]], rows_v.at[pl.ds(0, 128)],
                          sem)
    d1 = pltpu.async_copy(ys_hbm.at[pos2_v.at[1]], rows_v.at[pl.ds(128, 128)],
                          sem)
    d0.wait()
    d1.wait()
    pltpu.sync_copy(rows_v, y_hbm.at[pl.ds(wid * CHUNK, CHUNK)])


_sc_mesh = plsc.VectorSubcoreMesh(core_axis_name="c", subcore_axis_name="s")
_sc_params = pltpu.CompilerParams(needs_layout_passes=False)

_route = pl.kernel(
    _route_body,
    out_type=(
        jax.ShapeDtypeStruct((N, D), jnp.float32),
        jax.ShapeDtypeStruct((2 * NW, 128), jnp.int32),
        jax.ShapeDtypeStruct((NSTEP,), jnp.int32),
        jax.ShapeDtypeStruct((NSTEP,), jnp.int32),
        jax.ShapeDtypeStruct((NSTEP,), jnp.int32),
        jax.ShapeDtypeStruct((NSTEP,), jnp.int32),
        jax.ShapeDtypeStruct((NC, NS, K), jnp.int32),
    ),
    mesh=_sc_mesh,
    compiler_params=_sc_params,
    scratch_types=[
        pltpu.VMEM((K,), jnp.float32),
        pltpu.VMEM((CHUNK,), jnp.float32),
        pltpu.VMEM((CHUNK,), jnp.int32),
        pltpu.VMEM((K,), jnp.int32),
        pltpu.VMEM((K,), jnp.int32),
        pltpu.VMEM((NS, K), jnp.int32),
        pltpu.VMEM((2, 128), jnp.int32),
        pltpu.VMEM((16,), jnp.int32),
        pltpu.VMEM((16,), jnp.int32),
        pltpu.VMEM((SEGW,), jnp.int32),
        pltpu.VMEM((BPC,), jnp.int32),
        pltpu.VMEM((BPC,), jnp.int32),
        pltpu.VMEM((BPC,), jnp.int32),
        pltpu.VMEM((4, PPC), jnp.int32),
        pltpu.VMEM((CHUNK, D), jnp.float32),
        pltpu.SemaphoreType.DMA,
        pltpu.SemaphoreType.DMA,
    ],
)

_unsort = pl.kernel(
    _unsort_body,
    out_type=jax.ShapeDtypeStruct((N, D), jnp.float32),
    mesh=_sc_mesh,
    compiler_params=_sc_params,
    scratch_types=[
        pltpu.VMEM((2, 128), jnp.int32),
        pltpu.VMEM((CHUNK, D), jnp.float32),
        pltpu.SemaphoreType.DMA,
    ],
)


@jax.jit
def _run(u, x, part, means, devs_bf16):
    xs, pos, pb, pe, plo, phi, _ = _route(u, x, part)
    ys = pl.pallas_call(
        _gemm_body,
        grid_spec=pltpu.PrefetchScalarGridSpec(
            num_scalar_prefetch=4,
            grid=(NSTEP,),
            in_specs=[
                pl.BlockSpec((BLK, D), lambda g, pb, pe, plo, phi: (pb[g], 0)),
                pl.BlockSpec((K, D, D), lambda g, *_: (0, 0, 0)),
                pl.BlockSpec((K, D), lambda g, *_: (0, 0)),
            ],
            out_specs=pl.BlockSpec((BLK, D),
                                   lambda g, pb, pe, plo, phi: (pb[g], 0)),
        ),
        out_shape=jax.ShapeDtypeStruct((N, D), jnp.float32),
    )(pb, pe, plo, phi, xs, devs_bf16, means)
    return _unsort(ys, pos)


def kernel(z, means, devs, mix_partition):
    u = z[:, 0]
    x = z[:, 1:]
    return _run(u, x, mix_partition, means, devs.astype(jnp.bfloat16))
